# baseline (speedup 1.0000x reference)
import functools

import jax
import jax.numpy as jnp
from jax import lax
from jax.experimental import pallas as pl
from jax.experimental.pallas import tpu as pltpu
from jax.experimental.pallas import tpu_sc as plsc

_NTILES = 32
_ROWS1 = 2048
_COLS1 = 1600
_RPT = _ROWS1 // _NTILES
_RCHUNK = 8


def _sc_dsum1_kernel(p_ref, t_ref, out_ref, pbuf, tbuf, acc, sem_p, sem_t):
    wid = lax.axis_index("s") * 2 + lax.axis_index("c")
    base = wid * _RPT

    for j in range(_COLS1 // 16):
        acc[pl.ds(j * 16, 16)] = jnp.zeros((16,), jnp.float32)

    def step(k, carry):
        row = base + k * _RCHUNK
        cp = pltpu.async_copy(p_ref.at[pl.ds(row, _RCHUNK)], pbuf, sem_p)
        ct = pltpu.async_copy(t_ref.at[pl.ds(row, _RCHUNK)], tbuf, sem_t)
        cp.wait()
        ct.wait()

        def col(j, c2):
            a = acc[pl.ds(j * 16, 16)]
            for r in range(_RCHUNK):
                d = pbuf[r, pl.ds(j * 16, 16)] - tbuf[r, pl.ds(j * 16, 16)]
                a = a + d * d
            acc[pl.ds(j * 16, 16)] = a
            return c2

        return lax.fori_loop(0, _COLS1 // 16, col, carry)

    lax.fori_loop(0, _RPT // _RCHUNK, step, 0)
    pltpu.sync_copy(acc, out_ref.at[wid])


def _sc_dsum1(p, t):
    p = p.reshape(_ROWS1, _COLS1)
    t = t.reshape(_ROWS1, _COLS1)
    mesh = plsc.VectorSubcoreMesh(core_axis_name="c", subcore_axis_name="s")
    f = pl.kernel(
        _sc_dsum1_kernel,
        mesh=mesh,
        out_type=jax.ShapeDtypeStruct((_NTILES, _COLS1), jnp.float32),
        scratch_types=[
            pltpu.VMEM((_RCHUNK, _COLS1), jnp.float32),
            pltpu.VMEM((_RCHUNK, _COLS1), jnp.float32),
            pltpu.VMEM((_COLS1,), jnp.float32),
            pltpu.SemaphoreType.DMA,
            pltpu.SemaphoreType.DMA,
        ],
    )
    return f(p, t).reshape(8, 4, _COLS1)


def _mask_kernel(bid_ref, bb_ref, m_ref, *, S):
    ss = S * S
    box = jnp.floor(bb_ref[...] * S).astype(jnp.int32)
    xc = box[:, 0:1]
    yc = box[:, 1:2]
    wd = box[:, 2:3]
    ht = box[:, 3:4]
    xl = jnp.maximum(xc - wd // 2, 0)
    yt = jnp.maximum(yc - ht // 2, 0)
    xr = jnp.minimum(xc + wd // 2, S - 1)
    yd = jnp.minimum(yc + ht // 2, S - 1)
    w = (xr - xl + 1).astype(jnp.float32)
    h = (yd - yt + 1).astype(jnp.float32)
    idx = jax.lax.broadcasted_iota(jnp.int32, (1, ss), 1)
    xflat = idx % S
    yflat = idx // S
    dx = xflat.astype(jnp.float32) - xc.astype(jnp.float32)
    dy = yflat.astype(jnp.float32) - yc.astype(jnp.float32)
    arg = dx * dx / (w * w) + dy * dy / (h * h)
    g = jnp.exp(-arg)
    inside = ((xflat >= xl) & (xflat <= xr)
              & (yflat >= yt) & (yflat <= yd))
    g = jnp.where(inside, g, 0.0)
    bid = bid_ref[...]
    for b in range(8):
        gb = jnp.where(bid == b, g, 0.0)
        m_ref[b] = jnp.max(gb, axis=0, keepdims=True)


def _dsum_kernel(p_ref, t_ref, d_ref):
    c = pl.program_id(1)
    d = p_ref[...] - t_ref[...]
    s = jnp.sum(d * d, axis=1, keepdims=True)

    @pl.when(c == 0)
    def _():
        d_ref[...] = s

    @pl.when(c != 0)
    def _():
        d_ref[...] += s


def _combine_kernel(m0_ref, d0_ref, m1_ref, d1_ref, o_ref):
    m0 = m0_ref[...]
    r0 = jnp.sum(m0 * m0 * d0_ref[...])
    sm0 = jnp.sum(m0)
    m1 = m1_ref[...]
    d1 = jnp.sum(d1_ref[...], axis=1)[:, None, :]
    r1 = jnp.sum(m1 * m1 * d1)
    sm1 = jnp.sum(m1)
    acc = r0 / (256.0 * sm0) + r1 / (256.0 * sm1)
    o_ref[0, 0] = 0.5 * acc


def _masks(batch_idx, bboxes, S):
    bid = batch_idx.astype(jnp.int32).reshape(64, 1)
    return pl.pallas_call(
        functools.partial(_mask_kernel, S=S),
        out_shape=jax.ShapeDtypeStruct((8, 1, S * S), jnp.float32),
    )(bid, bboxes)


def _dsum(p, t, cb):
    B, C, S, _ = p.shape
    ss = S * S
    p = p.reshape(B, C, ss)
    t = t.reshape(B, C, ss)
    grid = (B, C // cb)
    return pl.pallas_call(
        _dsum_kernel,
        grid=grid,
        in_specs=[
            pl.BlockSpec((1, cb, ss), lambda b, c: (b, c, 0)),
            pl.BlockSpec((1, cb, ss), lambda b, c: (b, c, 0)),
        ],
        out_specs=pl.BlockSpec((1, 1, ss), lambda b, c: (b, 0, 0)),
        out_shape=jax.ShapeDtypeStruct((B, 1, ss), jnp.float32),
    )(p, t)


@jax.jit
def kernel(y_pred0, y_pred1, y_true0, y_true1, batch_idx, cls, bboxes):
    d1 = _sc_dsum1(y_pred1, y_true1)
    d0 = _dsum(y_pred0, y_true0, 256)
    m0 = _masks(batch_idx, bboxes, 80)
    m1 = _masks(batch_idx, bboxes, 40)
    out = pl.pallas_call(
        _combine_kernel,
        out_shape=jax.ShapeDtypeStruct((1, 1), jnp.float32),
        out_specs=pl.BlockSpec(memory_space=pltpu.SMEM),
    )(m0, d0, m1, d1)
    return out[0, 0]


# speedup vs baseline: 1.2842x; 1.2842x over previous
import jax
import jax.numpy as jnp
from jax import lax
from jax.experimental import pallas as pl
from jax.experimental.pallas import tpu as pltpu
from jax.experimental.pallas import tpu_sc as plsc


def _sc_mask_scale(wid, bidv, bbv, gxbuf, mbuf, out_ref, *, S, scale_base):
    half_rows = S // 2
    seg = half_rows * S
    b = (wid - scale_base) // 2
    half = (wid - scale_base) % 2
    y0 = half * half_rows

    for j in range(seg // 16):
        mbuf[pl.ds(j * 16, 16)] = jnp.zeros((16,), jnp.float32)

    lanes = jnp.arange(16, dtype=jnp.int32)
    sf = jnp.float32(S)

    def box(i, carry):
        bid_i = bidv[pl.ds(i, 16)][0]

        @pl.when(bid_i == b)
        def _():
            xc = (bbv[pl.ds(4 * i, 16)][0] * sf).astype(jnp.int32)
            yc = (bbv[pl.ds(4 * i + 1, 16)][0] * sf).astype(jnp.int32)
            wd = (bbv[pl.ds(4 * i + 2, 16)][0] * sf).astype(jnp.int32)
            ht = (bbv[pl.ds(4 * i + 3, 16)][0] * sf).astype(jnp.int32)
            xl = jnp.maximum(xc - wd // 2, 0)
            yt = jnp.maximum(yc - ht // 2, 0)
            xr = jnp.minimum(xc + wd // 2, S - 1)
            yd = jnp.minimum(yc + ht // 2, S - 1)
            w = (xr - xl + 1).astype(jnp.float32)
            h = (yd - yt + 1).astype(jnp.float32)
            xcg = xc.astype(jnp.float32)
            ycg = yc.astype(jnp.float32)
            wwv = jnp.full((16,), w, jnp.float32) * w
            hhv = jnp.full((16,), h, jnp.float32) * h
            for ci in range(S // 16):
                xs = lanes + (ci * 16)
                dxv = xs.astype(jnp.float32) - xcg
                gx = jnp.exp(-(dxv * dxv) / wwv)
                gx = jnp.where((xs >= xl) & (xs <= xr), gx, 0.0)
                gxbuf[pl.ds(ci * 16, 16)] = gx

            y_lo = jnp.maximum(yt, y0)
            y_hi = jnp.minimum(yd, y0 + half_rows - 1)

            def row(y, c2):
                dyf = y.astype(jnp.float32) - ycg
                dyv = jnp.full((16,), dyf, jnp.float32)
                gy = jnp.exp(-(dyv * dyv) / hhv)
                off = (y - y0) * S
                for ci in range(S // 16):
                    cur = mbuf[pl.ds(off + ci * 16, 16)]
                    gxc = gxbuf[pl.ds(ci * 16, 16)]
                    mbuf[pl.ds(off + ci * 16, 16)] = jnp.maximum(cur, gy * gxc)
                return c2

            lax.fori_loop(y_lo, y_hi + 1, row, 0)

        return carry

    lax.fori_loop(0, 64, box, 0)
    pltpu.sync_copy(mbuf.at[pl.ds(0, seg)],
                    out_ref.at[pl.ds(b * (2 * seg) + half * seg, seg)])


def _sc_mask_kernel(bid_ref, bb_ref, m0_ref, m1_ref, bidv, bbv, gxbuf, mbuf):
    wid = lax.axis_index("s") * 2 + lax.axis_index("c")
    pltpu.sync_copy(bid_ref, bidv)
    pltpu.sync_copy(bb_ref, bbv)

    def scale0():
        _sc_mask_scale(wid, bidv, bbv, gxbuf, mbuf, m0_ref, S=80, scale_base=0)

    def scale1():
        _sc_mask_scale(wid, bidv, bbv, gxbuf, mbuf, m1_ref, S=40, scale_base=16)

    lax.cond(wid < 16, scale0, scale1)


def _sc_masks(batch_idx, bboxes):
    bid = jnp.pad(batch_idx.astype(jnp.int32), (0, 16))
    bb = jnp.pad(bboxes.reshape(256), (0, 16))
    mesh = plsc.VectorSubcoreMesh(core_axis_name="c", subcore_axis_name="s")
    f = pl.kernel(
        _sc_mask_kernel,
        mesh=mesh,
        out_type=[
            jax.ShapeDtypeStruct((8 * 6400,), jnp.float32),
            jax.ShapeDtypeStruct((8 * 1600,), jnp.float32),
        ],
        scratch_types=[
            pltpu.VMEM((80,), jnp.int32),
            pltpu.VMEM((272,), jnp.float32),
            pltpu.VMEM((80,), jnp.float32),
            pltpu.VMEM((3200,), jnp.float32),
        ],
    )
    m0, m1 = f(bid, bb)
    return m0.reshape(8, 1, 6400), m1.reshape(8, 1, 1600)


def _dsum_kernel(p_ref, t_ref, d_ref):
    c = pl.program_id(1)
    d = p_ref[...] - t_ref[...]
    s = jnp.sum(d * d, axis=1, keepdims=True)

    @pl.when(c == 0)
    def _():
        d_ref[...] = s

    @pl.when(c != 0)
    def _():
        d_ref[...] += s


def _combine_kernel(m0_ref, d0_ref, m1_ref, d1_ref, o_ref):
    m0 = m0_ref[...]
    r0 = jnp.sum(m0 * m0 * d0_ref[...])
    sm0 = jnp.sum(m0)
    m1 = m1_ref[...]
    r1 = jnp.sum(m1 * m1 * d1_ref[...])
    sm1 = jnp.sum(m1)
    acc = r0 / (256.0 * sm0) + r1 / (256.0 * sm1)
    o_ref[0, 0] = 0.5 * acc


def _dsum(p, t, cb):
    B, C, S, _ = p.shape
    ss = S * S
    p = p.reshape(B, C, ss)
    t = t.reshape(B, C, ss)
    grid = (B, C // cb)
    return pl.pallas_call(
        _dsum_kernel,
        grid=grid,
        in_specs=[
            pl.BlockSpec((1, cb, ss), lambda b, c: (b, c, 0)),
            pl.BlockSpec((1, cb, ss), lambda b, c: (b, c, 0)),
        ],
        out_specs=pl.BlockSpec((1, 1, ss), lambda b, c: (b, 0, 0)),
        out_shape=jax.ShapeDtypeStruct((B, 1, ss), jnp.float32),
    )(p, t)


@jax.jit
def kernel(y_pred0, y_pred1, y_true0, y_true1, batch_idx, cls, bboxes):
    m0, m1 = _sc_masks(batch_idx, bboxes)
    d0 = _dsum(y_pred0, y_true0, 256)
    d1 = _dsum(y_pred1, y_true1, 256)
    out = pl.pallas_call(
        _combine_kernel,
        out_shape=jax.ShapeDtypeStruct((1, 1), jnp.float32),
        out_specs=pl.BlockSpec(memory_space=pltpu.SMEM),
    )(m0, d0, m1, d1)
    return out[0, 0]


# speedup vs baseline: 1.2863x; 1.0016x over previous
import jax
import jax.numpy as jnp
from jax import lax
from jax.experimental import pallas as pl
from jax.experimental.pallas import tpu as pltpu
from jax.experimental.pallas import tpu_sc as plsc


def _sc_mask_scale(wid, bidv, bbv, gxbuf, mbuf, out_ref, *, S, scale_base):
    half_rows = S // 2
    seg = half_rows * S
    b = (wid - scale_base) // 2
    half = (wid - scale_base) % 2
    y0 = half * half_rows

    for j in range(seg // 16):
        mbuf[pl.ds(j * 16, 16)] = jnp.zeros((16,), jnp.float32)

    lanes = jnp.arange(16, dtype=jnp.int32)
    sf = jnp.float32(S)

    def box(i, carry):
        bid_i = bidv[pl.ds(i, 16)][0]

        @pl.when(bid_i == b)
        def _():
            xc = (bbv[pl.ds(4 * i, 16)][0] * sf).astype(jnp.int32)
            yc = (bbv[pl.ds(4 * i + 1, 16)][0] * sf).astype(jnp.int32)
            wd = (bbv[pl.ds(4 * i + 2, 16)][0] * sf).astype(jnp.int32)
            ht = (bbv[pl.ds(4 * i + 3, 16)][0] * sf).astype(jnp.int32)
            xl = jnp.maximum(xc - wd // 2, 0)
            yt = jnp.maximum(yc - ht // 2, 0)
            xr = jnp.minimum(xc + wd // 2, S - 1)
            yd = jnp.minimum(yc + ht // 2, S - 1)
            w = (xr - xl + 1).astype(jnp.float32)
            h = (yd - yt + 1).astype(jnp.float32)
            xcg = xc.astype(jnp.float32)
            ycg = yc.astype(jnp.float32)
            wwv = jnp.full((16,), w, jnp.float32) * w
            hhv = jnp.full((16,), h, jnp.float32) * h
            for ci in range(S // 16):
                xs = lanes + (ci * 16)
                dxv = xs.astype(jnp.float32) - xcg
                gx = jnp.exp(-(dxv * dxv) / wwv)
                gx = jnp.where((xs >= xl) & (xs <= xr), gx, 0.0)
                gxbuf[pl.ds(ci * 16, 16)] = gx

            y_lo = jnp.maximum(yt, y0)
            y_hi = jnp.minimum(yd, y0 + half_rows - 1)

            def row(y, c2):
                dyf = y.astype(jnp.float32) - ycg
                dyv = jnp.full((16,), dyf, jnp.float32)
                gy = jnp.exp(-(dyv * dyv) / hhv)
                off = (y - y0) * S
                for ci in range(S // 16):
                    cur = mbuf[pl.ds(off + ci * 16, 16)]
                    gxc = gxbuf[pl.ds(ci * 16, 16)]
                    mbuf[pl.ds(off + ci * 16, 16)] = jnp.maximum(cur, gy * gxc)
                return c2

            lax.fori_loop(y_lo, y_hi + 1, row, 0)

        return carry

    lax.fori_loop(0, 64, box, 0)
    pltpu.sync_copy(mbuf.at[pl.ds(0, seg)],
                    out_ref.at[pl.ds(b * (2 * seg) + half * seg, seg)])


def _sc_mask_kernel(bid_ref, bb_ref, m0_ref, m1_ref, bidv, bbv, gxbuf, mbuf):
    wid = lax.axis_index("s") * 2 + lax.axis_index("c")
    pltpu.sync_copy(bid_ref, bidv)
    pltpu.sync_copy(bb_ref, bbv)

    def scale0():
        _sc_mask_scale(wid, bidv, bbv, gxbuf, mbuf, m0_ref, S=80, scale_base=0)

    def scale1():
        _sc_mask_scale(wid, bidv, bbv, gxbuf, mbuf, m1_ref, S=40, scale_base=16)

    lax.cond(wid < 16, scale0, scale1)


def _sc_masks(batch_idx, bboxes):
    bid = jnp.pad(batch_idx.astype(jnp.int32), (0, 16))
    bb = jnp.pad(bboxes.reshape(256), (0, 16))
    mesh = plsc.VectorSubcoreMesh(core_axis_name="c", subcore_axis_name="s")
    f = pl.kernel(
        _sc_mask_kernel,
        mesh=mesh,
        out_type=[
            jax.ShapeDtypeStruct((8 * 6400,), jnp.float32),
            jax.ShapeDtypeStruct((8 * 1600,), jnp.float32),
        ],
        scratch_types=[
            pltpu.VMEM((80,), jnp.int32),
            pltpu.VMEM((272,), jnp.float32),
            pltpu.VMEM((80,), jnp.float32),
            pltpu.VMEM((3200,), jnp.float32),
        ],
    )
    m0, m1 = f(bid, bb)
    return m0.reshape(8, 1, 6400), m1.reshape(8, 1, 1600)


def _dsum_kernel(p_ref, t_ref, d_ref):
    c = pl.program_id(1)
    d = p_ref[...] - t_ref[...]
    s = jnp.sum(d * d, axis=1, keepdims=True)

    @pl.when(c == 0)
    def _():
        d_ref[...] = s

    @pl.when(c != 0)
    def _():
        d_ref[...] += s


def _combine_kernel(m0_ref, d0_ref, m1_ref, d1_ref, o_ref):
    m0 = m0_ref[...]
    r0 = jnp.sum(m0 * m0 * d0_ref[...])
    sm0 = jnp.sum(m0)
    m1 = m1_ref[...]
    r1 = jnp.sum(m1 * m1 * d1_ref[...])
    sm1 = jnp.sum(m1)
    acc = r0 / (256.0 * sm0) + r1 / (256.0 * sm1)
    o_ref[0, 0] = 0.5 * acc


def _dsum(p, t, cb):
    B, C, S, _ = p.shape
    ss = S * S
    p = p.reshape(B, C, ss)
    t = t.reshape(B, C, ss)
    grid = (B, C // cb)
    return pl.pallas_call(
        _dsum_kernel,
        grid=grid,
        in_specs=[
            pl.BlockSpec((1, cb, ss), lambda b, c: (b, c, 0)),
            pl.BlockSpec((1, cb, ss), lambda b, c: (b, c, 0)),
        ],
        out_specs=pl.BlockSpec((1, 1, ss), lambda b, c: (b, 0, 0)),
        out_shape=jax.ShapeDtypeStruct((B, 1, ss), jnp.float32),
    )(p, t)


@jax.jit
def kernel(y_pred0, y_pred1, y_true0, y_true1, batch_idx, cls, bboxes):
    d0 = _dsum(y_pred0, y_true0, 256)
    d1 = _dsum(y_pred1, y_true1, 256)
    m0, m1 = _sc_masks(batch_idx, bboxes)
    out = pl.pallas_call(
        _combine_kernel,
        out_shape=jax.ShapeDtypeStruct((1, 1), jnp.float32),
        out_specs=pl.BlockSpec(memory_space=pltpu.SMEM),
    )(m0, d0, m1, d1)
    return out[0, 0]


# speedup vs baseline: 1.2875x; 1.0009x over previous
import jax
import jax.numpy as jnp
from jax import lax
from jax.experimental import pallas as pl
from jax.experimental.pallas import tpu as pltpu
from jax.experimental.pallas import tpu_sc as plsc


def _sc_mask_scale(wid, bidv, bbv, gxbuf, mbuf, out_ref, *, S, nsplit,
                   scale_base):
    half_rows = S // nsplit
    seg = half_rows * S
    b = (wid - scale_base) // nsplit
    half = (wid - scale_base) % nsplit
    y0 = half * half_rows

    for j in range(seg // 16):
        mbuf[pl.ds(j * 16, 16)] = jnp.zeros((16,), jnp.float32)

    lanes = jnp.arange(16, dtype=jnp.int32)
    sf = jnp.float32(S)

    def box(i, carry):
        bid_i = bidv[pl.ds(i, 16)][0]

        @pl.when(bid_i == b)
        def _():
            xc = (bbv[pl.ds(4 * i, 16)][0] * sf).astype(jnp.int32)
            yc = (bbv[pl.ds(4 * i + 1, 16)][0] * sf).astype(jnp.int32)
            wd = (bbv[pl.ds(4 * i + 2, 16)][0] * sf).astype(jnp.int32)
            ht = (bbv[pl.ds(4 * i + 3, 16)][0] * sf).astype(jnp.int32)
            xl = jnp.maximum(xc - wd // 2, 0)
            yt = jnp.maximum(yc - ht // 2, 0)
            xr = jnp.minimum(xc + wd // 2, S - 1)
            yd = jnp.minimum(yc + ht // 2, S - 1)
            w = (xr - xl + 1).astype(jnp.float32)
            h = (yd - yt + 1).astype(jnp.float32)
            xcg = xc.astype(jnp.float32)
            ycg = yc.astype(jnp.float32)
            wwv = jnp.full((16,), w, jnp.float32) * w
            hhv = jnp.full((16,), h, jnp.float32) * h
            for ci in range(S // 16):
                xs = lanes + (ci * 16)
                dxv = xs.astype(jnp.float32) - xcg
                gx = jnp.exp(-(dxv * dxv) / wwv)
                gx = jnp.where((xs >= xl) & (xs <= xr), gx, 0.0)
                gxbuf[pl.ds(ci * 16, 16)] = gx

            y_lo = jnp.maximum(yt, y0)
            y_hi = jnp.minimum(yd, y0 + half_rows - 1)

            def row(y, c2):
                dyf = y.astype(jnp.float32) - ycg
                dyv = jnp.full((16,), dyf, jnp.float32)
                gy = jnp.exp(-(dyv * dyv) / hhv)
                off = (y - y0) * S
                for ci in range(S // 16):
                    cur = mbuf[pl.ds(off + ci * 16, 16)]
                    gxc = gxbuf[pl.ds(ci * 16, 16)]
                    mbuf[pl.ds(off + ci * 16, 16)] = jnp.maximum(cur, gy * gxc)
                return c2

            lax.fori_loop(y_lo, y_hi + 1, row, 0)

        return carry

    lax.fori_loop(0, 64, box, 0)
    pltpu.sync_copy(mbuf.at[pl.ds(0, seg)],
                    out_ref.at[pl.ds(b * (nsplit * seg) + half * seg, seg)])


def _sc_mask_kernel(bid_ref, bb_ref, m0_ref, m1_ref, bidv, bbv, gxbuf, mbuf):
    wid = lax.axis_index("s") * 2 + lax.axis_index("c")
    pltpu.sync_copy(bid_ref, bidv)
    pltpu.sync_copy(bb_ref, bbv)

    _sc_mask_scale(wid, bidv, bbv, gxbuf, mbuf, m0_ref, S=80, nsplit=4,
                   scale_base=0)

    def scale1():
        _sc_mask_scale(wid, bidv, bbv, gxbuf, mbuf, m1_ref, S=40, nsplit=2,
                       scale_base=0)

    def noop():
        pass

    lax.cond(wid < 16, scale1, noop)


def _sc_masks(batch_idx, bboxes):
    bid = jnp.pad(batch_idx.astype(jnp.int32), (0, 16))
    bb = jnp.pad(bboxes.reshape(256), (0, 16))
    mesh = plsc.VectorSubcoreMesh(core_axis_name="c", subcore_axis_name="s")
    f = pl.kernel(
        _sc_mask_kernel,
        mesh=mesh,
        out_type=[
            jax.ShapeDtypeStruct((8 * 6400,), jnp.float32),
            jax.ShapeDtypeStruct((8 * 1600,), jnp.float32),
        ],
        scratch_types=[
            pltpu.VMEM((80,), jnp.int32),
            pltpu.VMEM((272,), jnp.float32),
            pltpu.VMEM((80,), jnp.float32),
            pltpu.VMEM((3200,), jnp.float32),
        ],
    )
    m0, m1 = f(bid, bb)
    return m0.reshape(8, 1, 6400), m1.reshape(8, 1, 1600)


def _dsum_kernel(p_ref, t_ref, d_ref):
    c = pl.program_id(1)
    d = p_ref[...] - t_ref[...]
    s = jnp.sum(d * d, axis=1, keepdims=True)

    @pl.when(c == 0)
    def _():
        d_ref[...] = s

    @pl.when(c != 0)
    def _():
        d_ref[...] += s


def _combine_kernel(m0_ref, d0_ref, m1_ref, d1_ref, o_ref):
    m0 = m0_ref[...]
    r0 = jnp.sum(m0 * m0 * d0_ref[...])
    sm0 = jnp.sum(m0)
    m1 = m1_ref[...]
    r1 = jnp.sum(m1 * m1 * d1_ref[...])
    sm1 = jnp.sum(m1)
    acc = r0 / (256.0 * sm0) + r1 / (256.0 * sm1)
    o_ref[0, 0] = 0.5 * acc


def _dsum(p, t, cb):
    B, C, S, _ = p.shape
    ss = S * S
    p = p.reshape(B, C, ss)
    t = t.reshape(B, C, ss)
    grid = (B, C // cb)
    return pl.pallas_call(
        _dsum_kernel,
        grid=grid,
        in_specs=[
            pl.BlockSpec((1, cb, ss), lambda b, c: (b, c, 0)),
            pl.BlockSpec((1, cb, ss), lambda b, c: (b, c, 0)),
        ],
        out_specs=pl.BlockSpec((1, 1, ss), lambda b, c: (b, 0, 0)),
        out_shape=jax.ShapeDtypeStruct((B, 1, ss), jnp.float32),
    )(p, t)


@jax.jit
def kernel(y_pred0, y_pred1, y_true0, y_true1, batch_idx, cls, bboxes):
    d0 = _dsum(y_pred0, y_true0, 256)
    d1 = _dsum(y_pred1, y_true1, 256)
    m0, m1 = _sc_masks(batch_idx, bboxes)
    out = pl.pallas_call(
        _combine_kernel,
        out_shape=jax.ShapeDtypeStruct((1, 1), jnp.float32),
        out_specs=pl.BlockSpec(memory_space=pltpu.SMEM),
    )(m0, d0, m1, d1)
    return out[0, 0]
